# Initial kernel scaffold; baseline (speedup 1.0000x reference)
import jax
import jax.numpy as jnp
from jax.experimental import pallas as pl


def kernel(t, x, edge_index, hyperedges):
    raise NotImplementedError("write your pallas kernel here")


# speedup vs baseline: 81.4957x; 81.4957x over previous
import functools

import jax
import jax.numpy as jnp
from jax import lax
from jax.experimental import pallas as pl
from jax.experimental.pallas import tpu as pltpu
from jax.experimental.pallas import tpu_sc as plsc

NC = 2
NS = 16
NW = NC * NS
L = 16
CH = 2048
CROWS = CH // 128


def _pad_flat(a, per_tile):
    total = NW * per_tile
    return jnp.concatenate([a, jnp.zeros((total - a.shape[0],), jnp.int32)])


def _make_sc_kernel(n, ept, hpt, zpad):
    e_chunks = ept // CH
    h_chunks = hpt // CH

    mesh = plsc.VectorSubcoreMesh(
        core_axis_name="c", subcore_axis_name="s", num_cores=NC,
        num_subcores=NS)

    @functools.partial(
        pl.kernel,
        out_type=jax.ShapeDtypeStruct((NC, zpad), jnp.float32),
        mesh=mesh,
        scratch_types=[
            pltpu.VMEM((n,), jnp.float32),
            pltpu.VMEM((CH,), jnp.int32),
            pltpu.VMEM((CH,), jnp.int32),
            pltpu.VMEM((CH,), jnp.int32),
            pltpu.VMEM((CH,), jnp.float32),
            pltpu.VMEM((CH,), jnp.float32),
            pltpu.VMEM((CH,), jnp.float32),
            pltpu.MemorySpace.VMEM_SHARED((zpad,), jnp.float32),
        ],
        compiler_params=pltpu.CompilerParams(needs_layout_passes=False),
    )
    def sc_kernel(x_hbm, src_hbm, dst_hbm, h1_hbm, h2_hbm, h3_hbm, z_hbm,
                  out_hbm, x_v, ia, ib, ic, va, vb, vc, acc):
        c = lax.axis_index("c")
        s = lax.axis_index("s")
        wid = c * NS + s

        @pl.when(s == 0)
        def _zero():
            pltpu.sync_copy(z_hbm, acc)

        pltpu.sync_copy(x_hbm, x_v)
        plsc.subcore_barrier()

        def edge_chunk(k, carry):
            b0 = wid * ept + k * CH
            pltpu.sync_copy(src_hbm.at[pl.ds(b0, CH)], ia)
            pltpu.sync_copy(dst_hbm.at[pl.ds(b0, CH)], ib)

            def vec(j, carry2):
                sl = pl.ds(j * L, L)
                si = ia[sl]
                di = ib[sl]
                xs = plsc.load_gather(x_v, [si])
                xd = plsc.load_gather(x_v, [di])
                va[sl] = -0.1 * (xd - xs)
                return carry2
            lax.fori_loop(0, CH // L, vec, carry)
            pltpu.sync_copy(va, acc.at[ib], add=True)
            return carry
        lax.fori_loop(0, e_chunks, edge_chunk, 0)

        def hyper_chunk(k, carry):
            b0 = wid * hpt + k * CH
            pltpu.sync_copy(h1_hbm.at[pl.ds(b0, CH)], ia)
            pltpu.sync_copy(h2_hbm.at[pl.ds(b0, CH)], ib)
            pltpu.sync_copy(h3_hbm.at[pl.ds(b0, CH)], ic)

            def vec(j, carry2):
                sl = pl.ds(j * L, L)
                i1 = ia[sl]
                i2 = ib[sl]
                i3 = ic[sl]
                x1 = plsc.load_gather(x_v, [i1])
                x2 = plsc.load_gather(x_v, [i2])
                x3 = plsc.load_gather(x_v, [i3])
                p = x2 * x3
                va[sl] = 0.9 * (p - x1 * x1)
                vb[sl] = 0.9 * (p - x2 * x2)
                vc[sl] = 0.9 * (p - x3 * x3)
                return carry2
            lax.fori_loop(0, CH // L, vec, carry)
            pltpu.sync_copy(va, acc.at[ia], add=True)
            pltpu.sync_copy(vb, acc.at[ib], add=True)
            pltpu.sync_copy(vc, acc.at[ic], add=True)
            return carry
        lax.fori_loop(0, h_chunks, hyper_chunk, 0)

        plsc.subcore_barrier()

        @pl.when(s == 0)
        def _out():
            pltpu.sync_copy(acc, out_hbm.at[c])

    return sc_kernel


def _sum2_body(parts_ref, o_ref):
    o_ref[...] = parts_ref[0, :] + parts_ref[1, :]


def kernel(t, x, edge_index, hyperedges):
    del t
    n = x.shape[0]
    e = edge_index.shape[1]
    h = hyperedges.shape[0]

    ept = -(-e // (NW * CH)) * CH
    hpt = -(-h // (NW * CH)) * CH
    zpad = -(-n // (NS * 128)) * (NS * 128)

    ei = edge_index.astype(jnp.int32)
    he = hyperedges.astype(jnp.int32)
    src2 = _pad_flat(ei[0], ept)
    dst2 = _pad_flat(ei[1], ept)
    h1 = _pad_flat(he[:, 0], hpt)
    h2 = _pad_flat(he[:, 1], hpt)
    h3 = _pad_flat(he[:, 2], hpt)
    z = jnp.zeros((zpad,), jnp.float32)

    sc_kernel = _make_sc_kernel(n, ept, hpt, zpad)
    parts = sc_kernel(x, src2, dst2, h1, h2, h3, z)

    summed = pl.pallas_call(
        _sum2_body,
        out_shape=jax.ShapeDtypeStruct((zpad,), jnp.float32),
    )(parts)
    return summed[:n]
